# baseline (speedup 1.0000x reference)
import functools

import jax
import jax.numpy as jnp
from jax import lax
from jax.experimental import pallas as pl
from jax.experimental.pallas import tpu as pltpu
from jax.experimental.pallas import tpu_sc as plsc

BATCH = 16384
D = 32
NC = 2
NS = 16
NW = NC * NS
BPW = BATCH // NW
CH = 128
NCH = BPW // CH
G = 16

N_USERS = 1000000
N_MOVIES = 100000
SCN = 286720
TCN = N_USERS - SCN
SC_PT = SCN // NW
CK = 896
NCHK = SC_PT // CK
NBUF = 3
BLK = 40960
OFF = SCN // BLK


def _proj_body(tT_ref, w_ref, bias_ref, out_ref):
    out_ref[...] = jnp.sum(tT_ref[...] * w_ref[...], axis=0) + bias_ref[0, 0]


def _project(tT, w_col, bias, off_blocks, n_out):
    grid = (n_out + BLK - 1) // BLK
    return pl.pallas_call(
        _proj_body,
        grid=(grid,),
        in_specs=[
            pl.BlockSpec((D, BLK), lambda i: (0, i + off_blocks)),
            pl.BlockSpec((D, 1), lambda i: (0, 0)),
            pl.BlockSpec((1, 1), lambda i: (0, 0)),
        ],
        out_specs=pl.BlockSpec((BLK,), lambda i: (i,)),
        out_shape=jax.ShapeDtypeStruct((n_out,), jnp.float32),
        compiler_params=pltpu.CompilerParams(
            vmem_limit_bytes=100 * 1024 * 1024),
    )(tT, w_col, bias)


_mesh = plsc.VectorSubcoreMesh(core_axis_name="c", subcore_axis_name="s")


@functools.partial(
    pl.kernel,
    mesh=_mesh,
    out_type=jax.ShapeDtypeStruct((SCN,), jnp.float32),
    scratch_types=[
        pltpu.VMEM((D,), jnp.float32),
        pltpu.VMEM((D, CK), jnp.float32),
        pltpu.VMEM((D, CK), jnp.float32),
        pltpu.VMEM((D, CK), jnp.float32),
        pltpu.VMEM((SC_PT,), jnp.float32),
        pltpu.SemaphoreType.DMA,
        pltpu.SemaphoreType.DMA,
        pltpu.SemaphoreType.DMA,
    ],
    compiler_params=pltpu.CompilerParams(use_tc_tiling_on_sc=True),
)
def _sc_proj(tT_hbm, w_hbm, out_hbm, wv, buf0, buf1, buf2, outv,
             sem0, sem1, sem2):
    wid = lax.axis_index("s") * NC + lax.axis_index("c")
    base = wid * SC_PT

    pltpu.sync_copy(w_hbm, wv)
    wlo = wv[pl.ds(0, G)]
    whi = wv[pl.ds(G, G)]

    bufs = (buf0, buf1, buf2)
    sems = (sem0, sem1, sem2)
    cps = [pltpu.async_copy(
        tT_hbm.at[:, pl.ds(base + i * CK, CK)], bufs[i], sems[i])
        for i in range(NBUF - 1)]
    for c in range(NCHK):
        if c + NBUF - 1 < NCHK:
            cps.append(pltpu.async_copy(
                tT_hbm.at[:, pl.ds(base + (c + NBUF - 1) * CK, CK)],
                bufs[(c + NBUF - 1) % NBUF], sems[(c + NBUF - 1) % NBUF]))
        cps[c].wait()
        buf = bufs[c % NBUF]
        out_base = c * CK

        def group(g, _):
            for k in range(2):
                sl = pl.ds((2 * g + k) * G, G)
                acc = jnp.zeros((G,), jnp.float32)
                for d in range(D):
                    ws = wlo[d] if d < G else whi[d - G]
                    acc = acc + buf[d, sl] * ws
                outv[pl.ds(out_base + (2 * g + k) * G, G)] = acc
            return 0

        lax.fori_loop(0, CK // G // 2, group, 0)

    pltpu.sync_copy(outv, out_hbm.at[pl.ds(base, SC_PT)])


@functools.partial(
    pl.kernel,
    mesh=_mesh,
    out_type=jax.ShapeDtypeStruct((BATCH,), jnp.float32),
    scratch_types=[
        pltpu.VMEM((BPW,), jnp.int32),
        pltpu.VMEM((BPW,), jnp.int32),
        pltpu.VMEM((BPW,), jnp.int32),
        pltpu.VMEM((BPW,), jnp.int32),
        pltpu.VMEM((BPW,), jnp.float32),
        pltpu.VMEM((BPW,), jnp.float32),
        pltpu.VMEM((BPW,), jnp.float32),
        pltpu.VMEM((BPW,), jnp.float32),
        pltpu.SemaphoreType.DMA,
    ],
    compiler_params=pltpu.CompilerParams(use_tc_tiling_on_sc=False),
)
def _sc_combine(u_hbm, m_hbm, psc_hbm, ptc_hbm, pm_hbm, out_hbm,
                uix, six, tix, mix, gs, gt, gm, outv, sem):
    wid = lax.axis_index("s") * NC + lax.axis_index("c")
    base = wid * BPW

    pltpu.sync_copy(u_hbm.at[pl.ds(base, BPW)], uix)
    pltpu.sync_copy(m_hbm.at[pl.ds(base, BPW)], mix)

    def split(g, _):
        sl = pl.ds(g * G, G)
        uv = uix[sl]
        sv = uv - SCN
        tix[sl] = jnp.where(sv < 0, sv + TCN, sv)
        wv = jnp.where(sv >= 0, sv, uv)
        wv = jnp.where(wv >= SCN, wv - SCN, wv)
        wv = jnp.where(wv >= SCN, wv - SCN, wv)
        six[sl] = wv
        return 0

    lax.fori_loop(0, BPW // G, split, 0)

    copies = []
    for j in range(NCH):
        sl = pl.ds(j * CH, CH)
        copies.append(pltpu.async_copy(psc_hbm.at[six.at[sl]], gs.at[sl], sem))
        copies.append(pltpu.async_copy(ptc_hbm.at[tix.at[sl]], gt.at[sl], sem))
        copies.append(pltpu.async_copy(pm_hbm.at[mix.at[sl]], gm.at[sl], sem))
    for cp in copies:
        cp.wait()

    def group(g, _):
        sl = pl.ds(g * G, G)
        uv = uix[sl]
        pu = jnp.where(uv < SCN, gs[sl], gt[sl])
        outv[sl] = pu + gm[sl]
        return 0

    lax.fori_loop(0, BPW // G, group, 0)

    pltpu.sync_copy(outv, out_hbm.at[pl.ds(base, BPW)])


def kernel(u, m, user_table, movie_table, W, b):
    wu = W[0, :D].reshape(D, 1).astype(jnp.float32)
    wm = W[0, D:].reshape(D, 1).astype(jnp.float32)
    zero = jnp.zeros((1, 1), jnp.float32)
    proj_m = _project(movie_table.T, wm, b.reshape(1, 1), 0, N_MOVIES)
    pu_sc = _sc_proj(user_table.T, W[0, :D].astype(jnp.float32))
    pu_tc = _project(user_table.T, wu, zero, OFF, TCN)
    out = _sc_combine(u.astype(jnp.int32), m.astype(jnp.int32),
                      pu_sc, pu_tc, proj_m)
    return out.reshape(BATCH, 1)


# speedup vs baseline: 1.0213x; 1.0213x over previous
import functools

import jax
import jax.numpy as jnp
from jax import lax
from jax.experimental import pallas as pl
from jax.experimental.pallas import tpu as pltpu
from jax.experimental.pallas import tpu_sc as plsc

BATCH = 16384
D = 32
NC = 2
NS = 16
NW = NC * NS
BPW = BATCH // NW
CH = 128
NCH = BPW // CH
G = 16
BLK = 65536


def _proj_body(tT_ref, w_ref, bias_ref, out_ref):
    out_ref[...] = jnp.sum(tT_ref[...] * w_ref[...], axis=0) + bias_ref[0, 0]


def _project(tT, w_col, bias, n_rows):
    grid = (n_rows + BLK - 1) // BLK
    return pl.pallas_call(
        _proj_body,
        grid=(grid,),
        in_specs=[
            pl.BlockSpec((D, BLK), lambda i: (0, i)),
            pl.BlockSpec((D, 1), lambda i: (0, 0)),
            pl.BlockSpec((1, 1), lambda i: (0, 0)),
        ],
        out_specs=pl.BlockSpec((BLK,), lambda i: (i,)),
        out_shape=jax.ShapeDtypeStruct((n_rows,), jnp.float32),
    )(tT, w_col, bias)


_mesh = plsc.VectorSubcoreMesh(core_axis_name="c", subcore_axis_name="s")


@functools.partial(
    pl.kernel,
    mesh=_mesh,
    out_type=jax.ShapeDtypeStruct((BATCH,), jnp.float32),
    scratch_types=[
        pltpu.VMEM((BPW,), jnp.int32),
        pltpu.VMEM((BPW,), jnp.int32),
        pltpu.VMEM((BPW,), jnp.float32),
        pltpu.VMEM((BPW,), jnp.float32),
        pltpu.VMEM((BPW,), jnp.float32),
        pltpu.SemaphoreType.DMA,
    ],
    compiler_params=pltpu.CompilerParams(use_tc_tiling_on_sc=False),
)
def _sc_lookup(u_hbm, m_hbm, pu_hbm, pm_hbm, out_hbm,
               uix, mix, gu, gm, outv, sem):
    wid = lax.axis_index("s") * NC + lax.axis_index("c")
    base = wid * BPW

    pltpu.sync_copy(u_hbm.at[pl.ds(base, BPW)], uix)
    pltpu.sync_copy(m_hbm.at[pl.ds(base, BPW)], mix)

    copies = []
    for j in range(NCH):
        sl = pl.ds(j * CH, CH)
        copies.append(pltpu.async_copy(pu_hbm.at[uix.at[sl]], gu.at[sl], sem))
        copies.append(pltpu.async_copy(pm_hbm.at[mix.at[sl]], gm.at[sl], sem))
    for cp in copies:
        cp.wait()

    def group(g, _):
        sl = pl.ds(g * G, G)
        outv[sl] = gu[sl] + gm[sl]
        return 0

    lax.fori_loop(0, BPW // G, group, 0)

    pltpu.sync_copy(outv, out_hbm.at[pl.ds(base, BPW)])


def kernel(u, m, user_table, movie_table, W, b):
    wu = W[0, :D].reshape(D, 1).astype(jnp.float32)
    wm = W[0, D:].reshape(D, 1).astype(jnp.float32)
    zero = jnp.zeros((1, 1), jnp.float32)
    proj_u = _project(user_table.T, wu, zero, user_table.shape[0])
    proj_m = _project(movie_table.T, wm, b.reshape(1, 1), movie_table.shape[0])
    out = _sc_lookup(u.astype(jnp.int32), m.astype(jnp.int32), proj_u, proj_m)
    return out.reshape(BATCH, 1)
